# baseline (speedup 1.0000x reference)
import functools

import jax
import jax.numpy as jnp
from jax import lax
from jax.experimental import pallas as pl
from jax.experimental.pallas import tpu as pltpu
from jax.experimental.pallas import tpu_sc as plsc

SIZE_OUT = 128
N_INPUTS = 64
BATCH = 16384
L = 16
NUM_CORES = 2
NUM_SUBCORES = 16
NW = NUM_CORES * NUM_SUBCORES

S_SC = 2048
ROWS = S_SC // NW
BLK = min(ROWS, 128)
NBLK = ROWS // BLK
UNROLL = 8

TC_ROWS = BATCH - S_SC
TC_BLK = 512

_mesh = plsc.VectorSubcoreMesh(core_axis_name="c", subcore_axis_name="s")


@functools.partial(
    pl.kernel,
    mesh=_mesh,
    compiler_params=pltpu.CompilerParams(needs_layout_passes=False),
    out_type=jax.ShapeDtypeStruct((S_SC, SIZE_OUT), jnp.float32),
    scratch_types=[
        pltpu.VMEM((BLK, N_INPUTS), jnp.float32),
        pltpu.VMEM((BLK, N_INPUTS), jnp.float32),
        pltpu.VMEM((BLK, SIZE_OUT), jnp.float32),
        pltpu.VMEM((BLK, SIZE_OUT), jnp.float32),
        pltpu.VMEM((N_INPUTS,), jnp.float32),
        pltpu.VMEM((N_INPUTS,), jnp.int32),
        pltpu.SemaphoreType.DMA,
        pltpu.SemaphoreType.DMA,
        pltpu.SemaphoreType.DMA,
        pltpu.SemaphoreType.DMA,
    ],
)
def _sc_scatter(
    x_hbm, w_hbm, ord_hbm, out_hbm,
    x0, x1, o0, o1, w_v, ord_v, si0, si1, so0, so1,
):
    wid = lax.axis_index("s") * NUM_CORES + lax.axis_index("c")
    base = wid * ROWS

    xb, ob = [x0, x1], [o0, o1]
    sin, sout = [si0, si1], [so0, so1]

    in_dma = [None] * NBLK
    out_dma = [None] * NBLK
    in_dma[0] = pltpu.async_copy(x_hbm.at[pl.ds(base, BLK)], xb[0], sin[0])
    pltpu.sync_copy(w_hbm, w_v)
    pltpu.sync_copy(ord_hbm, ord_v)

    n_in_chunks = N_INPUTS // L
    n_out_chunks = SIZE_OUT // L
    w_c = [w_v[pl.ds(c * L, L)] for c in range(n_in_chunks)]
    ord_c = [ord_v[pl.ds(c * L, L)] for c in range(n_in_chunks)]
    zeros = jnp.zeros((L,), jnp.float32)

    def zero_body(r, carry):
        for k in range(n_out_chunks):
            o0[r, pl.ds(k * L, L)] = zeros
            o1[r, pl.ds(k * L, L)] = zeros
        return carry

    lax.fori_loop(0, BLK, zero_body, 0)

    def make_row_loop(x_ref, o_ref):
        def row_body(t, carry):
            for u in range(UNROLL):
                r = t * UNROLL + u
                row_vec = jnp.full((L,), r, jnp.int32)
                for c in range(n_in_chunks):
                    val = x_ref[r, pl.ds(c * L, L)] * w_c[c]
                    plsc.store_scatter(o_ref, [row_vec, ord_c[c]], val)
            return carry

        return row_body

    for b in range(NBLK):
        i = b & 1
        if b + 1 < NBLK:
            in_dma[b + 1] = pltpu.async_copy(
                x_hbm.at[pl.ds(base + (b + 1) * BLK, BLK)], xb[(b + 1) & 1],
                sin[(b + 1) & 1],
            )
        in_dma[b].wait()
        if b >= 2:
            out_dma[b - 2].wait()
        lax.fori_loop(0, BLK // UNROLL, make_row_loop(xb[i], ob[i]), 0)
        out_dma[b] = pltpu.async_copy(
            ob[i], out_hbm.at[pl.ds(base + b * BLK, BLK)], sout[i]
        )

    if NBLK >= 2:
        out_dma[NBLK - 2].wait()
    out_dma[NBLK - 1].wait()


def _tc_body(xt_ref, w_ref, ord_ref, out_ref):
    ordv = ord_ref[...]
    wv = w_ref[...]
    cols = lax.broadcasted_iota(jnp.int32, (N_INPUTS, SIZE_OUT), 1)
    proj = jnp.where(ordv[:, None] == cols, wv[:, None], jnp.float32(0.0))
    out_ref[...] = lax.dot_general(
        xt_ref[...], proj, (((0,), (0,)), ((), ())),
        preferred_element_type=jnp.float32,
    )


_tc_scatter = pl.pallas_call(
    _tc_body,
    grid=(TC_ROWS // TC_BLK,),
    in_specs=[
        pl.BlockSpec((N_INPUTS, TC_BLK), lambda i: (0, S_SC // TC_BLK + i)),
        pl.BlockSpec((N_INPUTS,), lambda i: (0,)),
        pl.BlockSpec((N_INPUTS,), lambda i: (0,)),
    ],
    out_specs=pl.BlockSpec((TC_BLK, SIZE_OUT), lambda i: (S_SC // TC_BLK + i, 0)),
    out_shape=jax.ShapeDtypeStruct((BATCH, SIZE_OUT), jnp.float32),
)


def kernel(X_in, weights, input_node_order):
    w = weights.astype(jnp.float32)
    order = input_node_order.astype(jnp.int32)
    sc_part = _sc_scatter(X_in[:S_SC], w, order)
    tc_full = _tc_scatter(X_in.T, w, order)
    return lax.dynamic_update_slice(tc_full, sc_part, (0, 0))


# speedup vs baseline: 1.3685x; 1.3685x over previous
import functools

import jax
import jax.numpy as jnp
from jax import lax
from jax.experimental import pallas as pl
from jax.experimental.pallas import tpu as pltpu
from jax.experimental.pallas import tpu_sc as plsc

SIZE_OUT = 128
N_INPUTS = 64
BATCH = 16384
L = 16
NUM_CORES = 2
NUM_SUBCORES = 16
NW = NUM_CORES * NUM_SUBCORES

S_SC = 2048
ROWS = S_SC // NW
BLK = min(ROWS, 128)
NBLK = ROWS // BLK
UNROLL = 8

TC_ROWS = BATCH - S_SC
TC_BLK = 2048

_mesh = plsc.VectorSubcoreMesh(core_axis_name="c", subcore_axis_name="s")


@functools.partial(
    pl.kernel,
    mesh=_mesh,
    compiler_params=pltpu.CompilerParams(needs_layout_passes=False),
    out_type=jax.ShapeDtypeStruct((S_SC, SIZE_OUT), jnp.float32),
    scratch_types=[
        pltpu.VMEM((BLK, N_INPUTS), jnp.float32),
        pltpu.VMEM((BLK, N_INPUTS), jnp.float32),
        pltpu.VMEM((BLK, SIZE_OUT), jnp.float32),
        pltpu.VMEM((BLK, SIZE_OUT), jnp.float32),
        pltpu.VMEM((N_INPUTS,), jnp.float32),
        pltpu.VMEM((N_INPUTS,), jnp.int32),
        pltpu.SemaphoreType.DMA,
        pltpu.SemaphoreType.DMA,
        pltpu.SemaphoreType.DMA,
        pltpu.SemaphoreType.DMA,
    ],
)
def _sc_scatter(
    x_hbm, w_hbm, ord_hbm, out_hbm,
    x0, x1, o0, o1, w_v, ord_v, si0, si1, so0, so1,
):
    wid = lax.axis_index("s") * NUM_CORES + lax.axis_index("c")
    base = wid * ROWS

    xb, ob = [x0, x1], [o0, o1]
    sin, sout = [si0, si1], [so0, so1]

    in_dma = [None] * NBLK
    out_dma = [None] * NBLK
    in_dma[0] = pltpu.async_copy(x_hbm.at[pl.ds(base, BLK)], xb[0], sin[0])
    pltpu.sync_copy(w_hbm, w_v)
    pltpu.sync_copy(ord_hbm, ord_v)

    n_in_chunks = N_INPUTS // L
    n_out_chunks = SIZE_OUT // L
    w_c = [w_v[pl.ds(c * L, L)] for c in range(n_in_chunks)]
    ord_c = [ord_v[pl.ds(c * L, L)] for c in range(n_in_chunks)]
    zeros = jnp.zeros((L,), jnp.float32)

    def zero_body(r, carry):
        for k in range(n_out_chunks):
            o0[r, pl.ds(k * L, L)] = zeros
            o1[r, pl.ds(k * L, L)] = zeros
        return carry

    lax.fori_loop(0, BLK, zero_body, 0)

    def make_row_loop(x_ref, o_ref):
        def row_body(t, carry):
            for u in range(UNROLL):
                r = t * UNROLL + u
                row_vec = jnp.full((L,), r, jnp.int32)
                for c in range(n_in_chunks):
                    val = x_ref[r, pl.ds(c * L, L)] * w_c[c]
                    plsc.store_scatter(o_ref, [row_vec, ord_c[c]], val)
            return carry

        return row_body

    for b in range(NBLK):
        i = b & 1
        if b + 1 < NBLK:
            in_dma[b + 1] = pltpu.async_copy(
                x_hbm.at[pl.ds(base + (b + 1) * BLK, BLK)], xb[(b + 1) & 1],
                sin[(b + 1) & 1],
            )
        in_dma[b].wait()
        if b >= 2:
            out_dma[b - 2].wait()
        lax.fori_loop(0, BLK // UNROLL, make_row_loop(xb[i], ob[i]), 0)
        out_dma[b] = pltpu.async_copy(
            ob[i], out_hbm.at[pl.ds(base + b * BLK, BLK)], sout[i]
        )

    if NBLK >= 2:
        out_dma[NBLK - 2].wait()
    out_dma[NBLK - 1].wait()


def _tc_body(xt_ref, w_ref, ord_ref, out_ref):
    ordv = ord_ref[...]
    wv = w_ref[...]
    cols = lax.broadcasted_iota(jnp.int32, (N_INPUTS, SIZE_OUT), 1)
    proj = jnp.where(ordv[:, None] == cols, wv[:, None], jnp.float32(0.0))
    out_ref[...] = lax.dot_general(
        xt_ref[...], proj, (((0,), (0,)), ((), ())),
        preferred_element_type=jnp.float32,
    )


_tc_scatter = pl.pallas_call(
    _tc_body,
    grid=(TC_ROWS // TC_BLK,),
    in_specs=[
        pl.BlockSpec((N_INPUTS, TC_BLK), lambda i: (0, S_SC // TC_BLK + i)),
        pl.BlockSpec((N_INPUTS,), lambda i: (0,)),
        pl.BlockSpec((N_INPUTS,), lambda i: (0,)),
    ],
    out_specs=pl.BlockSpec((TC_BLK, SIZE_OUT), lambda i: (S_SC // TC_BLK + i, 0)),
    out_shape=jax.ShapeDtypeStruct((BATCH, SIZE_OUT), jnp.float32),
    compiler_params=pltpu.CompilerParams(fuse_transposed_lhs_in_matmul=True),
)


def kernel(X_in, weights, input_node_order):
    w = weights.astype(jnp.float32)
    order = input_node_order.astype(jnp.int32)
    sc_part = _sc_scatter(X_in[:S_SC], w, order)
    tc_full = _tc_scatter(X_in.T, w, order)
    return lax.dynamic_update_slice(tc_full, sc_part, (0, 0))
